# baseline (speedup 1.0000x reference)
import functools

import jax
import jax.numpy as jnp
from jax import lax
from jax.experimental import pallas as pl
from jax.experimental.pallas import tpu as pltpu
from jax.experimental.pallas import tpu_sc as plsc

_N_PREDICT = 3
_H = 224
_W = 224
_F = _N_PREDICT * _H * _W
_NW = 32
_C = _F // _NW


@functools.partial(jax.jit, static_argnums=(1,))
def _sc_broadcast(guess_flat, B):
    @functools.partial(
        pl.kernel,
        out_type=jax.ShapeDtypeStruct((B * _F,), jnp.float32),
        mesh=plsc.VectorSubcoreMesh(core_axis_name="c", subcore_axis_name="s"),
        scratch_types=[
            pltpu.VMEM((_C,), jnp.float32),
            pltpu.SemaphoreType.DMA,
        ],
    )
    def k(guess_hbm, out_hbm, buf_v, sem):
        wid = lax.axis_index("s") * 2 + lax.axis_index("c")
        base = wid * _C
        pltpu.sync_copy(guess_hbm.at[pl.ds(base, _C)], buf_v)
        copies = [
            pltpu.async_copy(buf_v, out_hbm.at[pl.ds(b * _F + base, _C)], sem)
            for b in range(B)
        ]
        for c in copies:
            c.wait()

    return k(guess_flat)


def kernel(era5_land, guess):
    B = era5_land.shape[0]
    out = _sc_broadcast(guess.reshape(_F), B)
    return out.reshape(B, _N_PREDICT, _H, _W)
